# baseline (speedup 1.0000x reference)
import functools

import jax
import jax.numpy as jnp
from jax import lax
from jax.experimental import pallas as pl
from jax.experimental.pallas import tpu as pltpu
from jax.experimental.pallas import tpu_sc as plsc

VOCAB = 49408
D = 512
N_CLS = 48
N_CTX = 8
MAX_SEG = 12
MAX_LEN = 256
IGNORE = -100
CLIP = 512

TAB_ROWS = 672
CTX_OFF = TAB_ROWS
PREFIX_ROW = CTX_OFF + N_CLS * N_CTX
ZERO_ROW = PREFIX_ROW + 1
SUFFIX_ROW = PREFIX_ROW + 2
NSRC = 1064


def _prep_body(labels_ref, table_ref, ctx_ref, pre_ref, suf_ref,
               gid_ref, s_ref):
    B, n = labels_ref.shape
    labels = labels_ref[...]
    lab_f = labels.astype(jnp.float32)

    km = lax.broadcasted_iota(jnp.int32, (n, n), 0)
    kn = lax.broadcasted_iota(jnp.int32, (n, n), 1)
    shift = (km == kn - 1).astype(jnp.float32)
    prev_f = jnp.dot(lab_f, shift, preferred_element_type=jnp.float32)

    pos = lax.broadcasted_iota(jnp.int32, (B, n), 1)
    chg = jnp.where(pos == 0, 1.0,
                    jnp.where(lab_f != prev_f, 1.0, 0.0))
    tri = (km <= kn).astype(jnp.float32)
    csum = jnp.dot(chg, tri, preferred_element_type=jnp.float32)

    idxs = [jnp.sum((csum <= float(j)).astype(jnp.int32), axis=1, keepdims=True)
            for j in range(MAX_SEG + 1)]
    labs, cnts = [], []
    for i in range(MAX_SEG):
        sel = (pos == idxs[i])
        labs.append(jnp.sum(jnp.where(sel, labels, 0), axis=1, keepdims=True))
        cnts.append(jnp.where(idxs[i] < n, idxs[i + 1] - idxs[i], 0))
    has_ign = jnp.sum((labels == IGNORE).astype(jnp.int32),
                      axis=1, keepdims=True) > 0

    p = lax.broadcasted_iota(jnp.int32, (B, MAX_LEN), 1)
    ps = jnp.clip(p - 1, 0, MAX_SEG * 21 - 1)
    seg = ps // 21
    r = ps - seg * 21
    cnt_sel = jnp.zeros((B, MAX_LEN), jnp.int32)
    lab_sel = jnp.zeros((B, MAX_LEN), jnp.int32)
    for i in range(MAX_SEG):
        mi = (seg == i)
        cnt_sel = jnp.where(mi, jnp.broadcast_to(cnts[i], (B, MAX_LEN)), cnt_sel)
        lab_sel = jnp.where(mi, jnp.broadcast_to(labs[i], (B, MAX_LEN)), lab_sel)
    gid = jnp.where(r < 10, r + seg * 13 + cnt_sel,
                    jnp.where(r < 10 + N_CTX,
                              CTX_OFF + lab_sel * N_CTX + (r - 10),
                              (r - 18) + lab_sel * 7 + 1))
    gid = jnp.where(p == 0, PREFIX_ROW, gid)
    gid = jnp.where(p >= 1 + MAX_SEG * 21, ZERO_ROW, gid)
    gid = jnp.where(has_ign, SUFFIX_ROW, gid)
    gid_ref[...] = gid

    s_ref[0:TAB_ROWS, :] = table_ref[...]
    s_ref[CTX_OFF:PREFIX_ROW, :] = ctx_ref[...].reshape(N_CLS * N_CTX, D)
    s_ref[PREFIX_ROW:NSRC, :] = jnp.zeros((NSRC - PREFIX_ROW, D), jnp.float32)
    s_ref[PREFIX_ROW:PREFIX_ROW + 1, :] = pre_ref[...]
    s_ref[SUFFIX_ROW:SUFFIX_ROW + 1, :] = suf_ref[...]


def _prep(labels, table, ctx, pre, suf):
    B = labels.shape[0]
    return pl.pallas_call(
        _prep_body,
        grid=(1,),
        out_shape=[
            jax.ShapeDtypeStruct((B, MAX_LEN), jnp.int32),
            jax.ShapeDtypeStruct((NSRC, D), jnp.float32),
        ],
        in_specs=[
            pl.BlockSpec((B, CLIP), lambda i: (0, 0)),
            pl.BlockSpec((TAB_ROWS, D), lambda i: (0, 0)),
            pl.BlockSpec((N_CLS, N_CTX, D), lambda i: (0, 0, 0)),
            pl.BlockSpec((1, D), lambda i: (0, 0)),
            pl.BlockSpec((1, D), lambda i: (0, 0)),
        ],
        out_specs=[
            pl.BlockSpec((B, MAX_LEN), lambda i: (0, 0)),
            pl.BlockSpec((NSRC, D), lambda i: (0, 0)),
        ],
    )(labels, table, ctx, pre, suf)


SZ = (32, 96)
OFF = (0, 32)
MH = 32


@functools.lru_cache(maxsize=None)
def _make_gather(nrows):
    info = plsc.get_sparse_core_info()
    nc = info.num_cores
    ns = info.num_subcores
    rp = nrows // (nc * ns)
    nch = len(SZ)
    mesh = plsc.VectorSubcoreMesh(core_axis_name="c", subcore_axis_name="s")

    scratch = ([pltpu.VMEM((rp,), jnp.int32)]
               + [pltpu.VMEM((s, D), jnp.float32) for s in SZ]
               + [pltpu.VMEM((MH, D), jnp.float32) for _ in range(2)]
               + [pltpu.SemaphoreType.DMA for _ in range(6)])

    @functools.partial(
        pl.kernel, mesh=mesh,
        out_type=[jax.ShapeDtypeStruct((nrows, D), jnp.float32),
                  jax.ShapeDtypeStruct((nrows, D), jnp.float32)],
        scratch_types=scratch,
    )
    def gather_k(s_hbm, gid_hbm, outp, outm, idx_v, *bs):
        sbufs = bs[:2]
        mbufs = bs[2:4]
        gsems = bs[4:6]
        psems = bs[6:8]
        msems = bs[8:10]
        wid = lax.axis_index("s") * nc + lax.axis_index("c")
        base = wid * rp
        pltpu.sync_copy(
            gid_hbm.at[wid // 2, pl.ds(pl.multiple_of((wid % 2) * rp, 8), rp)],
            idx_v)

        def gcopy(j):
            return pltpu.make_async_copy(
                s_hbm.at[idx_v.at[pl.ds(OFF[j], SZ[j])]],
                sbufs[j], gsems[j])

        def wpcopy(j):
            return pltpu.make_async_copy(
                sbufs[j],
                outp.at[pl.ds(base + OFF[j], SZ[j])], psems[j])

        mjobs = [(j, h * MH) for j in range(nch) for h in range(SZ[j] // MH)]

        def wmcopy(k):
            j, off = mjobs[k]
            return pltpu.make_async_copy(
                mbufs[k % 2],
                outm.at[pl.ds(base + OFF[j] + off, MH)], msems[k % 2])

        for j in range(nch):
            gcopy(j).start()
        k = 0
        for j in range(nch):
            gcopy(j).wait()
            wpcopy(j).start()
            for h in range(SZ[j] // MH):
                if k >= 2:
                    wmcopy(k - 2).wait()

                def row_mask(r, _, sb=sbufs[j], mb=mbufs[k % 2], off=h * MH):
                    for v in range(D // 16):
                        x = sb[off + r, pl.ds(v * 16, 16)]
                        mb[r, pl.ds(v * 16, 16)] = jnp.where(
                            x != 0.0, jnp.full((16,), 1.0, jnp.float32),
                            jnp.full((16,), 0.0, jnp.float32))
                    return _

                lax.fori_loop(0, MH, row_mask, 0)
                wmcopy(k).start()
                k += 1
        for j in range(nch):
            wpcopy(j).wait()
        for kk in range(max(0, k - 2), k):
            wmcopy(kk).wait()

    return gather_k


def kernel(last_clip_labels, batch_size, table, ctx, token_prefix, token_suffix):
    B = last_clip_labels.shape[0]
    labels = last_clip_labels.astype(jnp.int32)
    pre = token_prefix.reshape(1, D).astype(jnp.float32)
    suf = token_suffix.reshape(1, D).astype(jnp.float32)
    gid, src = _prep(labels, table, ctx, pre, suf)
    prompts, masks = _make_gather(B * MAX_LEN)(src, gid)
    return prompts.reshape(B, MAX_LEN, D), masks.reshape(B, MAX_LEN, D)


# speedup vs baseline: 1.0508x; 1.0508x over previous
import functools

import jax
import jax.numpy as jnp
from jax import lax
from jax.experimental import pallas as pl
from jax.experimental.pallas import tpu as pltpu
from jax.experimental.pallas import tpu_sc as plsc

VOCAB = 49408
D = 512
N_CLS = 48
N_CTX = 8
MAX_SEG = 12
MAX_LEN = 256
IGNORE = -100
CLIP = 512

TAB_ROWS = 672
CTX_OFF = TAB_ROWS
PREFIX_ROW = CTX_OFF + N_CLS * N_CTX
ZERO_ROW = PREFIX_ROW + 1
SUFFIX_ROW = PREFIX_ROW + 2
NSRC = 1064


def _prep_body(labels_ref, table_ref, ctx_ref, pre_ref, suf_ref,
               gid_ref, s_ref):
    B, n = labels_ref.shape
    labels = labels_ref[...]
    lab_f = labels.astype(jnp.float32)

    km = lax.broadcasted_iota(jnp.int32, (n, n), 0)
    kn = lax.broadcasted_iota(jnp.int32, (n, n), 1)
    shift = (km == kn - 1).astype(jnp.float32)
    prev_f = jnp.dot(lab_f, shift, preferred_element_type=jnp.float32)

    pos = lax.broadcasted_iota(jnp.int32, (B, n), 1)
    chg = jnp.where(pos == 0, 1.0,
                    jnp.where(lab_f != prev_f, 1.0, 0.0))
    tri = (km <= kn).astype(jnp.float32)
    csum = jnp.dot(chg, tri, preferred_element_type=jnp.float32)

    idxs = [jnp.sum((csum <= float(j)).astype(jnp.int32), axis=1, keepdims=True)
            for j in range(MAX_SEG + 1)]
    labs, cnts = [], []
    for i in range(MAX_SEG):
        sel = (pos == idxs[i])
        labs.append(jnp.sum(jnp.where(sel, labels, 0), axis=1, keepdims=True))
        cnts.append(jnp.where(idxs[i] < n, idxs[i + 1] - idxs[i], 0))
    has_ign = jnp.sum((labels == IGNORE).astype(jnp.int32),
                      axis=1, keepdims=True) > 0

    p = lax.broadcasted_iota(jnp.int32, (B, MAX_LEN), 1)
    ps = jnp.clip(p - 1, 0, MAX_SEG * 21 - 1)
    seg = ps // 21
    r = ps - seg * 21
    cnt_sel = jnp.zeros((B, MAX_LEN), jnp.int32)
    lab_sel = jnp.zeros((B, MAX_LEN), jnp.int32)
    for i in range(MAX_SEG):
        mi = (seg == i)
        cnt_sel = jnp.where(mi, jnp.broadcast_to(cnts[i], (B, MAX_LEN)), cnt_sel)
        lab_sel = jnp.where(mi, jnp.broadcast_to(labs[i], (B, MAX_LEN)), lab_sel)
    gid = jnp.where(r < 10, r + seg * 13 + cnt_sel,
                    jnp.where(r < 10 + N_CTX,
                              CTX_OFF + lab_sel * N_CTX + (r - 10),
                              (r - 18) + lab_sel * 7 + 1))
    gid = jnp.where(p == 0, PREFIX_ROW, gid)
    gid = jnp.where(p >= 1 + MAX_SEG * 21, ZERO_ROW, gid)
    gid = jnp.where(has_ign, SUFFIX_ROW, gid)
    gid_ref[...] = gid

    s_ref[0:TAB_ROWS, :] = table_ref[...]
    s_ref[CTX_OFF:PREFIX_ROW, :] = ctx_ref[...].reshape(N_CLS * N_CTX, D)
    s_ref[PREFIX_ROW:NSRC, :] = jnp.zeros((NSRC - PREFIX_ROW, D), jnp.float32)
    s_ref[PREFIX_ROW:PREFIX_ROW + 1, :] = pre_ref[...]
    s_ref[SUFFIX_ROW:SUFFIX_ROW + 1, :] = suf_ref[...]


def _prep(labels, table, ctx, pre, suf):
    B = labels.shape[0]
    return pl.pallas_call(
        _prep_body,
        grid=(1,),
        out_shape=[
            jax.ShapeDtypeStruct((B, MAX_LEN), jnp.int32),
            jax.ShapeDtypeStruct((NSRC, D), jnp.float32),
        ],
        in_specs=[
            pl.BlockSpec((B, CLIP), lambda i: (0, 0)),
            pl.BlockSpec((TAB_ROWS, D), lambda i: (0, 0)),
            pl.BlockSpec((N_CLS, N_CTX, D), lambda i: (0, 0, 0)),
            pl.BlockSpec((1, D), lambda i: (0, 0)),
            pl.BlockSpec((1, D), lambda i: (0, 0)),
        ],
        out_specs=[
            pl.BlockSpec((B, MAX_LEN), lambda i: (0, 0)),
            pl.BlockSpec((NSRC, D), lambda i: (0, 0)),
        ],
    )(labels, table, ctx, pre, suf)


CH = 64
MH = 32


@functools.lru_cache(maxsize=None)
def _make_gather(nrows):
    info = plsc.get_sparse_core_info()
    nc = info.num_cores
    ns = info.num_subcores
    rp = nrows // (nc * ns)
    nch = rp // CH
    mesh = plsc.VectorSubcoreMesh(core_axis_name="c", subcore_axis_name="s")

    scratch = ([pltpu.VMEM((rp,), jnp.int32)]
               + [pltpu.VMEM((CH, D), jnp.float32) for _ in range(2)]
               + [pltpu.VMEM((MH, D), jnp.float32) for _ in range(2)]
               + [pltpu.SemaphoreType.DMA for _ in range(6)])

    @functools.partial(
        pl.kernel, mesh=mesh,
        out_type=[jax.ShapeDtypeStruct((nrows, D), jnp.float32),
                  jax.ShapeDtypeStruct((nrows, D), jnp.float32)],
        scratch_types=scratch,
    )
    def gather_k(s_hbm, gid_hbm, outp, outm, idx_v, *bs):
        sbufs = bs[:2]
        mbufs = bs[2:4]
        gsems = bs[4:6]
        psems = bs[6:8]
        msems = bs[8:10]
        wid = lax.axis_index("s") * nc + lax.axis_index("c")
        base = wid * rp
        pltpu.sync_copy(
            gid_hbm.at[wid // 2, pl.ds(pl.multiple_of((wid % 2) * rp, 8), rp)],
            idx_v)

        def gcopy(j):
            return pltpu.make_async_copy(
                s_hbm.at[idx_v.at[pl.ds(j * CH, CH)]],
                sbufs[j], gsems[j])

        def wpcopy(j):
            return pltpu.make_async_copy(
                sbufs[j],
                outp.at[pl.ds(base + j * CH, CH)], psems[j])

        def wmcopy(j, h):
            return pltpu.make_async_copy(
                mbufs[h],
                outm.at[pl.ds(base + j * CH + h * MH, MH)], msems[h])

        for j in range(nch):
            gcopy(j).start()
        for j in range(nch):
            gcopy(j).wait()
            wpcopy(j).start()
            for h in range(CH // MH):
                if j > 0:
                    wmcopy(j - 1, h).wait()

                def row_mask(r, _, sb=sbufs[j], mb=mbufs[h], off=h * MH):
                    for v in range(D // 16):
                        x = sb[off + r, pl.ds(v * 16, 16)]
                        mb[r, pl.ds(v * 16, 16)] = jnp.where(
                            x != 0.0, jnp.full((16,), 1.0, jnp.float32),
                            jnp.full((16,), 0.0, jnp.float32))
                    return _

                lax.fori_loop(0, MH, row_mask, 0)
                wmcopy(j, h).start()
        for j in range(nch):
            wpcopy(j).wait()
        for h in range(CH // MH):
            wmcopy(nch - 1, h).wait()

    return gather_k


def kernel(last_clip_labels, batch_size, table, ctx, token_prefix, token_suffix):
    B = last_clip_labels.shape[0]
    labels = last_clip_labels.astype(jnp.int32)
    pre = token_prefix.reshape(1, D).astype(jnp.float32)
    suf = token_suffix.reshape(1, D).astype(jnp.float32)
    gid, src = _prep(labels, table, ctx, pre, suf)
    prompts, masks = _make_gather(B * MAX_LEN)(src, gid)
    return prompts.reshape(B, MAX_LEN, D), masks.reshape(B, MAX_LEN, D)
